# initial kernel scaffold (unmeasured)
import jax
import jax.numpy as jnp
from jax import lax
from jax.experimental import pallas as pl
from jax.experimental.pallas import tpu as pltpu

N_DEV = 4


def kernel(x, w_mat):
    m_per, k_dim = x.shape
    _, n = w_mat.shape
    n_per = n // N_DEV

    def body(x_ref, w_ref, out_ref, y_ref, comm_ref, amax_ref,
             send_sems, recv_sems, a_send_sems, a_recv_sems):
        me = lax.axis_index("i")

        barrier_sem = pltpu.get_barrier_semaphore()
        for kk in range(1, N_DEV):
            peer = lax.rem(me + kk, N_DEV)
            pl.semaphore_signal(barrier_sem, inc=1, device_id=(peer,),
                                device_id_type=pl.DeviceIdType.MESH)
        pl.semaphore_wait(barrier_sem, N_DEV - 1)

        x_val = x_ref[...].astype(jnp.bfloat16)
        sends = []
        amax_local = jnp.float32(0.0)

        for kk in range(1, N_DEV):
            t = lax.rem(me + kk, N_DEV)
            col0 = t * n_per
            w_chunk = pl.load(
                w_ref, (slice(None), pl.ds(col0, n_per))
            ).astype(jnp.bfloat16)
            y = jnp.dot(x_val, w_chunk, preferred_element_type=jnp.float32)
            y = jnp.maximum(y, 0.0)
            amax_local = jnp.maximum(amax_local, jnp.max(y))
            pl.store(y_ref, (slice(None), pl.ds(col0, n_per)),
                     y.astype(jnp.bfloat16))
            rdma = pltpu.make_async_remote_copy(
                src_ref=y_ref.at[:, pl.ds(col0, n_per)],
                dst_ref=comm_ref.at[me],
                send_sem=send_sems.at[t],
                recv_sem=recv_sems.at[me],
                device_id=(t,),
                device_id_type=pl.DeviceIdType.MESH,
            )
            rdma.start()
            sends.append(rdma)

        col0 = me * n_per
        w_chunk = pl.load(
            w_ref, (slice(None), pl.ds(col0, n_per))
        ).astype(jnp.bfloat16)
        y = jnp.dot(x_val, w_chunk, preferred_element_type=jnp.float32)
        y = jnp.maximum(y, 0.0)
        amax_local = jnp.maximum(amax_local, jnp.max(y))
        pl.store(comm_ref, (pl.ds(me, 1), slice(None), slice(None)),
                 y.astype(jnp.bfloat16)[None])

        pl.store(amax_ref, (pl.ds(me, 1), slice(None)),
                 jnp.full((1, 128), amax_local, jnp.float32))
        for kk in range(1, N_DEV):
            t = lax.rem(me + kk, N_DEV)
            a = pltpu.make_async_remote_copy(
                src_ref=amax_ref.at[me],
                dst_ref=amax_ref.at[me],
                send_sem=a_send_sems.at[t],
                recv_sem=a_recv_sems.at[me],
                device_id=(t,),
                device_id_type=pl.DeviceIdType.MESH,
            )
            a.start()
            sends.append(a)

        for kk in range(1, N_DEV):
            s = lax.rem(me + kk, N_DEV)
            recv = pltpu.make_async_remote_copy(
                src_ref=comm_ref.at[s],
                dst_ref=comm_ref.at[s],
                send_sem=send_sems.at[s],
                recv_sem=recv_sems.at[s],
                device_id=(s,),
                device_id_type=pl.DeviceIdType.MESH,
            )
            recv.wait_recv()
            a_recv = pltpu.make_async_remote_copy(
                src_ref=amax_ref.at[s],
                dst_ref=amax_ref.at[s],
                send_sem=a_send_sems.at[s],
                recv_sem=a_recv_sems.at[s],
                device_id=(s,),
                device_id_type=pl.DeviceIdType.MESH,
            )
            a_recv.wait_recv()

        g_amax = jnp.max(amax_ref[...])
        scale = g_amax / 127.0
        for j in range(N_DEV):
            v = comm_ref[j].astype(jnp.float32)
            q = jnp.clip(jnp.round(v / scale), -127.0, 127.0)
            out_ref[pl.ds(j * m_per, m_per), :] = q * scale

        for rdma in sends:
            rdma.wait_send()

    return pl.pallas_call(
        body,
        out_shape=jax.ShapeDtypeStruct((N_DEV * m_per, n_per), jnp.float32),
        in_specs=[pl.BlockSpec(memory_space=pltpu.VMEM),
                  pl.BlockSpec(memory_space=pltpu.VMEM)],
        out_specs=pl.BlockSpec(memory_space=pltpu.VMEM),
        scratch_shapes=[
            pltpu.VMEM((m_per, n), jnp.bfloat16),
            pltpu.VMEM((N_DEV, m_per, n_per), jnp.bfloat16),
            pltpu.VMEM((N_DEV, 128), jnp.float32),
            pltpu.SemaphoreType.DMA((N_DEV,)),
            pltpu.SemaphoreType.DMA((N_DEV,)),
            pltpu.SemaphoreType.DMA((N_DEV,)),
            pltpu.SemaphoreType.DMA((N_DEV,)),
        ],
        compiler_params=pltpu.CompilerParams(collective_id=0),
    )(x, w_mat)


# baseline (device time: 66906 ns/iter reference)
import jax
import jax.numpy as jnp
from jax import lax
from jax.experimental import pallas as pl
from jax.experimental.pallas import tpu as pltpu

N_DEV = 4
N_BUF = 3
HALVES = 2


def kernel(x, w_mat):
    m_per, k_dim = x.shape
    _, n = w_mat.shape
    n_per = n // N_DEV
    n_half = n_per // HALVES
    n_steps = N_DEV * HALVES

    def body(x_ref, w_hbm, out_ref, y_ref, comm_ref, amax_ref, w_buf,
             copy_sems, send_sems, recv_sems, a_send_sems, a_recv_sems):
        me = lax.axis_index("i")

        order = [lax.rem(me + kk, N_DEV) for kk in range(1, N_DEV)] + [me]
        half_col = [order[i // HALVES] * n_per + (i % HALVES) * n_half
                    for i in range(n_steps)]

        def w_copy(i):
            return pltpu.make_async_copy(
                w_hbm.at[:, pl.ds(half_col[i], n_half)],
                w_buf.at[i % N_BUF], copy_sems.at[i % N_BUF])

        w_copies = [w_copy(0), w_copy(1)]
        w_copies[0].start()
        w_copies[1].start()

        barrier_sem = pltpu.get_barrier_semaphore()
        for kk in range(1, N_DEV):
            peer = lax.rem(me + kk, N_DEV)
            pl.semaphore_signal(barrier_sem, inc=1, device_id=(peer,),
                                device_id_type=pl.DeviceIdType.MESH)
        pl.semaphore_wait(barrier_sem, N_DEV - 1)

        x_val = x_ref[...].astype(jnp.bfloat16)
        sends = []
        amax_local = jnp.float32(0.0)

        for i in range(n_steps):
            dev_step = i // HALVES
            half = i % HALVES
            w_copies[i].wait()
            if i + 2 < n_steps:
                nxt = w_copy(i + 2)
                nxt.start()
                w_copies.append(nxt)
            w_chunk = w_buf[i % N_BUF].astype(jnp.bfloat16)
            y = jnp.dot(x_val, w_chunk, preferred_element_type=jnp.float32)
            y = jnp.maximum(y, 0.0)
            amax_local = jnp.maximum(amax_local, jnp.max(y))
            if dev_step < N_DEV - 1:
                y_ref[dev_step, :, pl.ds(half * n_half, n_half)] = (
                    y.astype(jnp.bfloat16))
                if half == HALVES - 1:
                    t = order[dev_step]
                    rdma = pltpu.make_async_remote_copy(
                        src_ref=y_ref.at[dev_step],
                        dst_ref=comm_ref.at[me],
                        send_sem=send_sems.at[t],
                        recv_sem=recv_sems.at[me],
                        device_id=(t,),
                        device_id_type=pl.DeviceIdType.MESH,
                    )
                    rdma.start()
                    sends.append(rdma)
            else:
                comm_ref[pl.ds(me, 1), :, pl.ds(half * n_half, n_half)] = (
                    y.astype(jnp.bfloat16)[None])

        amax_ref[pl.ds(me, 1), :] = jnp.full((1, 128), amax_local, jnp.float32)
        for kk in range(1, N_DEV):
            t = lax.rem(me + kk, N_DEV)
            a = pltpu.make_async_remote_copy(
                src_ref=amax_ref.at[me],
                dst_ref=amax_ref.at[me],
                send_sem=a_send_sems.at[t],
                recv_sem=a_recv_sems.at[me],
                device_id=(t,),
                device_id_type=pl.DeviceIdType.MESH,
            )
            a.start()
            sends.append(a)

        for kk in range(1, N_DEV):
            s = lax.rem(me + kk, N_DEV)
            recv = pltpu.make_async_remote_copy(
                src_ref=comm_ref.at[s],
                dst_ref=comm_ref.at[s],
                send_sem=send_sems.at[s],
                recv_sem=recv_sems.at[s],
                device_id=(s,),
                device_id_type=pl.DeviceIdType.MESH,
            )
            recv.wait_recv()
            a_recv = pltpu.make_async_remote_copy(
                src_ref=amax_ref.at[s],
                dst_ref=amax_ref.at[s],
                send_sem=a_send_sems.at[s],
                recv_sem=a_recv_sems.at[s],
                device_id=(s,),
                device_id_type=pl.DeviceIdType.MESH,
            )
            a_recv.wait_recv()

        g_amax = jnp.max(amax_ref[...])
        scale = g_amax / 127.0
        for j in range(N_DEV):
            v = comm_ref[j].astype(jnp.float32)
            q = jnp.clip(jnp.round(v / scale), -127.0, 127.0)
            out_ref[pl.ds(j * m_per, m_per), :] = q * scale

        for rdma in sends:
            rdma.wait_send()

    return pl.pallas_call(
        body,
        out_shape=jax.ShapeDtypeStruct((N_DEV * m_per, n_per), jnp.float32),
        in_specs=[pl.BlockSpec(memory_space=pltpu.VMEM),
                  pl.BlockSpec(memory_space=pl.ANY)],
        out_specs=pl.BlockSpec(memory_space=pltpu.VMEM),
        scratch_shapes=[
            pltpu.VMEM((N_DEV - 1, m_per, n_per), jnp.bfloat16),
            pltpu.VMEM((N_DEV, m_per, n_per), jnp.bfloat16),
            pltpu.VMEM((N_DEV, 128), jnp.float32),
            pltpu.VMEM((N_BUF, k_dim, n_half), jnp.float32),
            pltpu.SemaphoreType.DMA((N_BUF,)),
            pltpu.SemaphoreType.DMA((N_DEV,)),
            pltpu.SemaphoreType.DMA((N_DEV,)),
            pltpu.SemaphoreType.DMA((N_DEV,)),
            pltpu.SemaphoreType.DMA((N_DEV,)),
        ],
        compiler_params=pltpu.CompilerParams(
            collective_id=0, vmem_limit_bytes=64 * 1024 * 1024),
    )(x, w_mat)


# device time: 59171 ns/iter; 1.1307x vs baseline; 1.1307x over previous
import jax
import jax.numpy as jnp
from jax import lax
from jax.experimental import pallas as pl
from jax.experimental.pallas import tpu as pltpu

N_DEV = 4
HALVES = 2


def kernel(x, w_mat):
    m_per, k_dim = x.shape
    _, n = w_mat.shape
    n_per = n // N_DEV
    n_half = n_per // HALVES
    n_steps = N_DEV * HALVES

    def body(x_ref, w_hbm, out_ref, y_ref, comm_ref, amax_ref, amax_smem,
             send_sems, recv_sems, a_send_sems, a_recv_sems):
        me = lax.axis_index("i")

        barrier_sem = pltpu.get_barrier_semaphore()
        for kk in range(1, N_DEV):
            peer = lax.rem(me + kk, N_DEV)
            pl.semaphore_signal(barrier_sem, inc=1, device_id=(peer,),
                                device_id_type=pl.DeviceIdType.MESH)
        pl.semaphore_wait(barrier_sem, N_DEV - 1)

        x_val = x_ref[...].astype(jnp.bfloat16)
        amax_smem[0] = jnp.float32(0.0)

        def step_dev(i):
            return lax.rem(me + 1 + i // 2, N_DEV)

        def inner(indices, w_chunk_ref):
            i = indices[0]
            t = step_dev(i)
            half = i % HALVES
            dev_step = i // HALVES
            y = jnp.dot(x_val, w_chunk_ref[...].astype(jnp.bfloat16),
                        preferred_element_type=jnp.float32)
            y = jnp.maximum(y, 0.0)
            amax_smem[0] = jnp.maximum(amax_smem[0], jnp.max(y))
            y_bf = y.astype(jnp.bfloat16)

            @pl.when(dev_step < N_DEV - 1)
            def _():
                y_ref[dev_step, :, pl.ds(half * n_half, n_half)] = y_bf

            @pl.when(jnp.logical_and(dev_step < N_DEV - 1,
                                     half == HALVES - 1))
            def _():
                rdma = pltpu.make_async_remote_copy(
                    src_ref=y_ref.at[dev_step],
                    dst_ref=comm_ref.at[me],
                    send_sem=send_sems.at[t],
                    recv_sem=recv_sems.at[me],
                    device_id=(t,),
                    device_id_type=pl.DeviceIdType.MESH,
                )
                rdma.start()

            @pl.when(dev_step == N_DEV - 1)
            def _():
                comm_ref[pl.ds(me, 1), :, pl.ds(half * n_half, n_half)] = (
                    y_bf[None])

        pltpu.emit_pipeline(
            inner,
            grid=(n_steps,),
            in_specs=[pl.BlockSpec(
                (k_dim, n_half),
                lambda i: (0, step_dev(i) * HALVES + i % HALVES))],
            _explicit_indices=True,
        )(w_hbm)

        amax_ref[pl.ds(me, 1), :] = jnp.full((1, 128), amax_smem[0],
                                             jnp.float32)
        for kk in range(1, N_DEV):
            t = lax.rem(me + kk, N_DEV)
            a = pltpu.make_async_remote_copy(
                src_ref=amax_ref.at[me],
                dst_ref=amax_ref.at[me],
                send_sem=a_send_sems.at[t],
                recv_sem=a_recv_sems.at[me],
                device_id=(t,),
                device_id_type=pl.DeviceIdType.MESH,
            )
            a.start()

        for kk in range(1, N_DEV):
            s = lax.rem(me + kk, N_DEV)
            a_recv = pltpu.make_async_remote_copy(
                src_ref=amax_ref.at[s],
                dst_ref=amax_ref.at[s],
                send_sem=a_send_sems.at[s],
                recv_sem=a_recv_sems.at[s],
                device_id=(s,),
                device_id_type=pl.DeviceIdType.MESH,
            )
            a_recv.wait_recv()
        g_amax = jnp.max(amax_ref[...])
        inv_scale = 127.0 / g_amax
        scale = g_amax / 127.0

        def quant_block(j):
            v = comm_ref[j].astype(jnp.float32)
            q = jnp.clip(jnp.round(v * inv_scale), -127.0, 127.0)
            out_ref[pl.ds(j * m_per, m_per), :] = q * scale

        quant_block(me)
        for kk in range(1, N_DEV):
            s = lax.rem(me + kk, N_DEV)
            recv = pltpu.make_async_remote_copy(
                src_ref=comm_ref.at[s],
                dst_ref=comm_ref.at[s],
                send_sem=send_sems.at[s],
                recv_sem=recv_sems.at[s],
                device_id=(s,),
                device_id_type=pl.DeviceIdType.MESH,
            )
            recv.wait_recv()
            quant_block(s)

        for kk in range(1, N_DEV):
            t = lax.rem(me + kk, N_DEV)
            chunk_done = pltpu.make_async_remote_copy(
                src_ref=y_ref.at[0],
                dst_ref=comm_ref.at[me],
                send_sem=send_sems.at[t],
                recv_sem=recv_sems.at[me],
                device_id=(t,),
                device_id_type=pl.DeviceIdType.MESH,
            )
            chunk_done.wait_send()
            a_done = pltpu.make_async_remote_copy(
                src_ref=amax_ref.at[me],
                dst_ref=amax_ref.at[me],
                send_sem=a_send_sems.at[t],
                recv_sem=a_recv_sems.at[me],
                device_id=(t,),
                device_id_type=pl.DeviceIdType.MESH,
            )
            a_done.wait_send()

    return pl.pallas_call(
        body,
        out_shape=jax.ShapeDtypeStruct((N_DEV * m_per, n_per), jnp.float32),
        in_specs=[pl.BlockSpec(memory_space=pltpu.VMEM),
                  pl.BlockSpec(memory_space=pl.ANY)],
        out_specs=pl.BlockSpec(memory_space=pltpu.VMEM),
        scratch_shapes=[
            pltpu.VMEM((N_DEV - 1, m_per, n_per), jnp.bfloat16),
            pltpu.VMEM((N_DEV, m_per, n_per), jnp.bfloat16),
            pltpu.VMEM((N_DEV, 128), jnp.float32),
            pltpu.SMEM((1,), jnp.float32),
            pltpu.SemaphoreType.DMA((N_DEV,)),
            pltpu.SemaphoreType.DMA((N_DEV,)),
            pltpu.SemaphoreType.DMA((N_DEV,)),
            pltpu.SemaphoreType.DMA((N_DEV,)),
        ],
        compiler_params=pltpu.CompilerParams(
            collective_id=0, vmem_limit_bytes=64 * 1024 * 1024),
    )(x, w_mat)


# device time: 55270 ns/iter; 1.2105x vs baseline; 1.0706x over previous
import jax
import jax.numpy as jnp
from jax import lax
from jax.experimental import pallas as pl
from jax.experimental.pallas import tpu as pltpu

N_DEV = 4
HALVES = 2


def kernel(x, w_mat):
    m_per, k_dim = x.shape
    _, n = w_mat.shape
    n_per = n // N_DEV
    n_half = n_per // HALVES
    n_steps = N_DEV * HALVES

    def body(x_ref, w_hbm, out_hbm, y_ref, comm_ref, amax_ref, amax_smem,
             out_stage, send_sems, recv_sems, a_send_sems, a_recv_sems,
             out_sems):
        me = lax.axis_index("i")

        barrier_sem = pltpu.get_barrier_semaphore()
        for kk in range(1, N_DEV):
            peer = lax.rem(me + kk, N_DEV)
            pl.semaphore_signal(barrier_sem, inc=1, device_id=(peer,),
                                device_id_type=pl.DeviceIdType.MESH)
        pl.semaphore_wait(barrier_sem, N_DEV - 1)

        x_val = x_ref[...].astype(jnp.bfloat16)
        amax_smem[0] = jnp.float32(0.0)

        def step_dev(i):
            return lax.rem(me + 1 + i // 2, N_DEV)

        def inner(indices, w_chunk_ref):
            i = indices[0]
            t = step_dev(i)
            half = i % HALVES
            dev_step = i // HALVES
            y = jnp.dot(x_val, w_chunk_ref[...].astype(jnp.bfloat16),
                        preferred_element_type=jnp.float32)
            y = jnp.maximum(y, 0.0)
            amax_smem[0] = jnp.maximum(amax_smem[0], jnp.max(y))
            y_bf = y.astype(jnp.bfloat16)

            @pl.when(dev_step < N_DEV - 1)
            def _():
                y_ref[dev_step, :, pl.ds(half * n_half, n_half)] = y_bf
                rdma = pltpu.make_async_remote_copy(
                    src_ref=y_ref.at[dev_step, :, pl.ds(half * n_half,
                                                        n_half)],
                    dst_ref=comm_ref.at[me, :, pl.ds(half * n_half,
                                                     n_half)],
                    send_sem=send_sems.at[HALVES * t + half],
                    recv_sem=recv_sems.at[HALVES * me + half],
                    device_id=(t,),
                    device_id_type=pl.DeviceIdType.MESH,
                )
                rdma.start()

            @pl.when(dev_step == N_DEV - 1)
            def _():
                comm_ref[pl.ds(me, 1), :, pl.ds(half * n_half, n_half)] = (
                    y_bf[None])

        pltpu.emit_pipeline(
            inner,
            grid=(n_steps,),
            in_specs=[pl.BlockSpec(
                (k_dim, n_half),
                lambda i: (0, step_dev(i) * HALVES + i % HALVES))],
            _explicit_indices=True,
        )(w_hbm)

        amax_ref[pl.ds(me, 1), :] = jnp.full((1, 128), amax_smem[0],
                                             jnp.float32)
        for kk in range(1, N_DEV):
            t = lax.rem(me + kk, N_DEV)
            a = pltpu.make_async_remote_copy(
                src_ref=amax_ref.at[me],
                dst_ref=amax_ref.at[me],
                send_sem=a_send_sems.at[t],
                recv_sem=a_recv_sems.at[me],
                device_id=(t,),
                device_id_type=pl.DeviceIdType.MESH,
            )
            a.start()

        for kk in range(1, N_DEV):
            s = lax.rem(me + kk, N_DEV)
            a_recv = pltpu.make_async_remote_copy(
                src_ref=amax_ref.at[s],
                dst_ref=amax_ref.at[s],
                send_sem=a_send_sems.at[s],
                recv_sem=a_recv_sems.at[s],
                device_id=(s,),
                device_id_type=pl.DeviceIdType.MESH,
            )
            a_recv.wait_recv()
        g_amax = jnp.max(amax_ref[...])
        inv_scale = 127.0 / g_amax
        scale = g_amax / 127.0

        def quant_block(j, slot):
            v = comm_ref[j].astype(jnp.float32)
            q = jnp.clip(jnp.round(v * inv_scale), -127.0, 127.0)
            out_stage[slot] = q * scale
            cp = pltpu.make_async_copy(
                out_stage.at[slot],
                out_hbm.at[pl.ds(j * m_per, m_per), :],
                out_sems.at[slot])
            cp.start()

        quant_block(me, 0)
        for kk in range(1, N_DEV):
            s = lax.rem(me + kk, N_DEV)
            for h in range(HALVES):
                recv = pltpu.make_async_remote_copy(
                    src_ref=comm_ref.at[s, :, pl.ds(h * n_half, n_half)],
                    dst_ref=comm_ref.at[s, :, pl.ds(h * n_half, n_half)],
                    send_sem=send_sems.at[HALVES * s + h],
                    recv_sem=recv_sems.at[HALVES * s + h],
                    device_id=(s,),
                    device_id_type=pl.DeviceIdType.MESH,
                )
                recv.wait_recv()
            quant_block(s, kk)

        for slot in range(N_DEV):
            pltpu.make_async_copy(
                out_stage.at[slot],
                out_hbm.at[pl.ds(0, m_per), :],
                out_sems.at[slot]).wait()
        for kk in range(1, N_DEV):
            t = lax.rem(me + kk, N_DEV)
            for h in range(HALVES):
                chunk_done = pltpu.make_async_remote_copy(
                    src_ref=y_ref.at[0, :, pl.ds(h * n_half, n_half)],
                    dst_ref=comm_ref.at[me, :, pl.ds(h * n_half, n_half)],
                    send_sem=send_sems.at[HALVES * t + h],
                    recv_sem=recv_sems.at[HALVES * me + h],
                    device_id=(t,),
                    device_id_type=pl.DeviceIdType.MESH,
                )
                chunk_done.wait_send()
            a_done = pltpu.make_async_remote_copy(
                src_ref=amax_ref.at[me],
                dst_ref=amax_ref.at[me],
                send_sem=a_send_sems.at[t],
                recv_sem=a_recv_sems.at[me],
                device_id=(t,),
                device_id_type=pl.DeviceIdType.MESH,
            )
            a_done.wait_send()

    return pl.pallas_call(
        body,
        out_shape=jax.ShapeDtypeStruct((N_DEV * m_per, n_per), jnp.float32),
        in_specs=[pl.BlockSpec(memory_space=pltpu.VMEM),
                  pl.BlockSpec(memory_space=pl.ANY)],
        out_specs=pl.BlockSpec(memory_space=pl.ANY),
        scratch_shapes=[
            pltpu.VMEM((N_DEV - 1, m_per, n_per), jnp.bfloat16),
            pltpu.VMEM((N_DEV, m_per, n_per), jnp.bfloat16),
            pltpu.VMEM((N_DEV, 128), jnp.float32),
            pltpu.SMEM((1,), jnp.float32),
            pltpu.VMEM((N_DEV, m_per, n_per), jnp.float32),
            pltpu.SemaphoreType.DMA((HALVES * N_DEV,)),
            pltpu.SemaphoreType.DMA((HALVES * N_DEV,)),
            pltpu.SemaphoreType.DMA((N_DEV,)),
            pltpu.SemaphoreType.DMA((N_DEV,)),
            pltpu.SemaphoreType.DMA((N_DEV,)),
        ],
        compiler_params=pltpu.CompilerParams(
            collective_id=0, vmem_limit_bytes=64 * 1024 * 1024),
    )(x, w_mat)
